# Initial kernel scaffold; baseline (speedup 1.0000x reference)
import jax
import jax.numpy as jnp
from jax import lax
from jax.experimental import pallas as pl
from jax.experimental.pallas import tpu as pltpu
from jax.experimental.pallas import tpu_sc as plsc

ROWS = 16384
D = 64
K = 1024
BLK = 512
NBLK = ROWS // BLK

NC, NS = 2, 16
NW = NC * NS
BPW = ROWS // NW


def _tc_distance_argmin(z_ref, cb_ref, cn_ref, idx_ref, loss_ref):
    i = pl.program_id(0)
    z = z_ref[...]
    cb = cb_ref[...]
    s = lax.dot_general(z, cb, (((1,), (1,)), ((), ())),
                        preferred_element_type=jnp.float32)
    zn = jnp.sum(z ** 2, axis=1, keepdims=True)
    d = zn - 2.0 * s + cn_ref[...]
    m = jnp.min(d, axis=1, keepdims=True)
    col = lax.broadcasted_iota(jnp.int32, d.shape, 1)
    idx = jnp.min(jnp.where(d <= m, col, K), axis=1)
    idx_ref[...] = idx.astype(jnp.int32)

    part = jnp.sum(m)

    @pl.when(i == 0)
    def _init():
        loss_ref[0, 0] = part

    @pl.when(i > 0)
    def _acc():
        loss_ref[0, 0] = loss_ref[0, 0] + part

    @pl.when(i == NBLK - 1)
    def _finish():
        loss_ref[0, 0] = loss_ref[0, 0] * (1.25 / (ROWS * D))


def _sc_gather_body(cb_hbm, idx_hbm, out_hbm, idx_v, rows_v, sem):
    wid = lax.axis_index("s") * NC + lax.axis_index("c")
    base = wid * BPW
    pltpu.sync_copy(idx_hbm.at[pl.ds(base, BPW)], idx_v)
    pltpu.async_copy(cb_hbm.at[idx_v], rows_v, sem).wait()
    pltpu.sync_copy(rows_v, out_hbm.at[pl.ds(base, BPW)])


_sc_gather = pl.kernel(
    _sc_gather_body,
    out_type=jax.ShapeDtypeStruct((ROWS, D), jnp.float32),
    mesh=plsc.VectorSubcoreMesh(core_axis_name="c", subcore_axis_name="s"),
    scratch_types=[
        pltpu.VMEM((BPW,), jnp.int32),
        pltpu.VMEM((BPW, D), jnp.float32),
        pltpu.SemaphoreType.DMA,
    ],
)


@jax.jit
def kernel(z, codebook):
    B, N, Dd = z.shape
    zf = z.reshape(-1, Dd)
    cn = jnp.sum(codebook ** 2, axis=1)[None, :]

    idx, loss = pl.pallas_call(
        _tc_distance_argmin,
        grid=(NBLK,),
        in_specs=[
            pl.BlockSpec((BLK, D), lambda i: (i, 0)),
            pl.BlockSpec((K, D), lambda i: (0, 0)),
            pl.BlockSpec((1, K), lambda i: (0, 0)),
        ],
        out_specs=[
            pl.BlockSpec((BLK,), lambda i: (i,)),
            pl.BlockSpec((1, 1), lambda i: (0, 0)),
        ],
        out_shape=[
            jax.ShapeDtypeStruct((ROWS,), jnp.int32),
            jax.ShapeDtypeStruct((1, 1), jnp.float32),
        ],
    )(zf, codebook, cn)

    zq = _sc_gather(codebook, idx)
    return zq.reshape(B, N, Dd), loss[0, 0], idx.reshape(B, N)


# speedup vs baseline: 1.0081x; 1.0081x over previous
import functools

import jax
import jax.numpy as jnp
from jax import lax
from jax.experimental import pallas as pl
from jax.experimental.pallas import tpu as pltpu
from jax.experimental.pallas import tpu_sc as plsc

ROWS = 16384
D = 64
K = 1024
BLK = 512
NBLK = ROWS // BLK

NC, NS = 2, 16
NW = NC * NS
BPW = ROWS // NW


def _tc_distance_argmin(z_ref, cb_ref, cn_ref, idx_ref, loss_ref):
    i = pl.program_id(0)
    z = z_ref[...]
    cb = cb_ref[...]
    s = lax.dot_general(z, cb, (((1,), (1,)), ((), ())),
                        preferred_element_type=jnp.float32)
    zn = jnp.sum(z ** 2, axis=1, keepdims=True)
    d = zn - 2.0 * s + cn_ref[...]
    m = jnp.min(d, axis=1, keepdims=True)
    col = lax.broadcasted_iota(jnp.int32, d.shape, 1)
    idx = jnp.min(jnp.where(d <= m, col, K), axis=1)
    idx_ref[...] = idx.astype(jnp.int32)

    part = jnp.sum(m, axis=0, keepdims=True)

    @pl.when(i == 0)
    def _init():
        loss_ref[...] = part

    @pl.when(i > 0)
    def _acc():
        loss_ref[...] = loss_ref[...] + part

    @pl.when(i == NBLK - 1)
    def _finish():
        loss_ref[...] = loss_ref[...] * (1.25 / (ROWS * D))


def _sc_gather_body(cb_hbm, idx_hbm, out_hbm, idx_v, rows_v, sem):
    wid = lax.axis_index("s") * NC + lax.axis_index("c")
    base = wid * BPW
    pltpu.sync_copy(idx_hbm.at[pl.ds(base, BPW)], idx_v)
    pltpu.async_copy(cb_hbm.at[idx_v], rows_v, sem).wait()
    pltpu.sync_copy(rows_v, out_hbm.at[pl.ds(base, BPW)])


@functools.cache
def _make_sc_gather():
    return pl.kernel(
        _sc_gather_body,
        out_type=jax.ShapeDtypeStruct((ROWS, D), jnp.float32),
        mesh=plsc.VectorSubcoreMesh(core_axis_name="c", subcore_axis_name="s",
                                    num_cores=NC, num_subcores=NS),
        scratch_types=[
            pltpu.VMEM((BPW,), jnp.int32),
            pltpu.VMEM((BPW, D), jnp.float32),
            pltpu.SemaphoreType.DMA,
        ],
        compiler_params=pltpu.CompilerParams(use_tc_tiling_on_sc=False),
    )


@jax.jit
def kernel(z, codebook):
    B, N, Dd = z.shape
    zf = z.reshape(-1, Dd)
    cn = jnp.sum(codebook ** 2, axis=1)[None, :]

    idx, loss = pl.pallas_call(
        _tc_distance_argmin,
        grid=(NBLK,),
        in_specs=[
            pl.BlockSpec((BLK, D), lambda i: (i, 0)),
            pl.BlockSpec((K, D), lambda i: (0, 0)),
            pl.BlockSpec((1, K), lambda i: (0, 0)),
        ],
        out_specs=[
            pl.BlockSpec((BLK,), lambda i: (i,)),
            pl.BlockSpec((1, 1), lambda i: (0, 0)),
        ],
        out_shape=[
            jax.ShapeDtypeStruct((ROWS,), jnp.int32),
            jax.ShapeDtypeStruct((1, 1), jnp.float32),
        ],
    )(zf, codebook, cn)

    zq = _make_sc_gather()(codebook, idx)
    return zq.reshape(B, N, Dd), loss[0, 0], idx.reshape(B, N)
